# baseline (speedup 1.0000x reference)
import functools

import jax
import jax.numpy as jnp
from jax import lax
from jax.experimental import pallas as pl
from jax.experimental.pallas import tpu as pltpu
from jax.experimental.pallas import tpu_sc as plsc

IN_LEN = 16384
OUT_LEN = 1024
NNZ = 262144
BATCH = 16
LANES = 16

NUM_CORES = 2
NUM_SUBCORES = 16
NW = NUM_CORES * NUM_SUBCORES
PER_W = NNZ // NW
SEG = 128
CHUNK = 1024
NSEG = CHUNK // SEG
NCHUNK = PER_W // CHUNK
GROUPS = CHUNK // LANES
NBUF = 4
ACC = OUT_LEN * BATCH


def _sc_body(xt_hbm, rows_hbm, cols_hbm, w_hbm, out_hbm,
             rows_v, cols_v, w_v, gath_v, acc_v, sem_idx, sem_gat, sem_out):
    cid = lax.axis_index("c")
    sid = lax.axis_index("s")
    wid = sid * NUM_CORES + cid
    iota_l = lax.iota(jnp.int32, LANES)

    zero = jnp.zeros((LANES,), jnp.float32)

    def zb(i, c):
        acc_v[pl.ds(i * LANES, LANES)] = zero
        return c

    lax.fori_loop(0, OUT_LEN, zb, 0)

    def issue_idx(t):
        b = t % NBUF
        nnz_base = pl.multiple_of(wid * PER_W + t * CHUNK, CHUNK)
        seg_base = pl.multiple_of(nnz_base // SEG, NSEG)
        return [
            pltpu.async_copy(rows_hbm.at[pl.ds(seg_base, NSEG)],
                             rows_v.at[b], sem_idx.at[b]),
            pltpu.async_copy(cols_hbm.at[pl.ds(nnz_base, CHUNK)],
                             cols_v.at[b], sem_idx.at[b]),
            pltpu.async_copy(w_hbm.at[pl.ds(nnz_base, CHUNK)],
                             w_v.at[b], sem_idx.at[b]),
        ]

    def issue_gat(t):
        b = t % NBUF
        return [
            pltpu.async_copy(xt_hbm.at[rows_v.at[b, s]],
                             gath_v.at[b, pl.ds(s * SEG, SEG)], sem_gat.at[b])
            for s in range(NSEG)
        ]

    def accumulate(t):
        b = t % NBUF

        def grp(g, c):
            gb = g * LANES
            for j in range(LANES):
                pos = gb + j
                splat = jnp.full((LANES,), pos, jnp.int32)
                wj = plsc.load_gather(w_v.at[b], [splat])
                cj = plsc.load_gather(cols_v.at[b], [splat])
                idx = cj * LANES + iota_l
                plsc.addupdate_scatter(acc_v, [idx],
                                       wj * gath_v[b, pos, :])
            return c

        lax.fori_loop(0, GROUPS, grp, 0)

    idx_d = {0: issue_idx(0), 1: issue_idx(1)}
    for d in idx_d[0]:
        d.wait()
    gat_d = {0: issue_gat(0)}
    for t in range(NCHUNK):
        if t + 2 < NCHUNK:
            idx_d[t + 2] = issue_idx(t + 2)
        if t + 1 < NCHUNK:
            for d in idx_d[t + 1]:
                d.wait()
            gat_d[t + 1] = issue_gat(t + 1)
        for d in gat_d[t]:
            d.wait()
        accumulate(t)

    pltpu.async_copy(acc_v, out_hbm.at[wid], sem_out).wait()


_sc_call = pl.kernel(
    _sc_body,
    out_type=jax.ShapeDtypeStruct((NW, ACC), jnp.float32),
    mesh=plsc.VectorSubcoreMesh(core_axis_name="c", subcore_axis_name="s"),
    compiler_params=pltpu.CompilerParams(needs_layout_passes=False,
                                         use_tc_tiling_on_sc=False),
    scratch_types=[
        pltpu.VMEM((NBUF, NSEG, SEG), jnp.int32),
        pltpu.VMEM((NBUF, CHUNK), jnp.int32),
        pltpu.VMEM((NBUF, CHUNK), jnp.float32),
        pltpu.VMEM((NBUF, CHUNK, LANES), jnp.float32),
        pltpu.VMEM((ACC,), jnp.float32),
        pltpu.SemaphoreType.DMA((NBUF,)),
        pltpu.SemaphoreType.DMA((NBUF,)),
        pltpu.SemaphoreType.DMA,
    ],
)


def _combine_body(parts_ref, bias_ref, out_ref):
    out_ref[...] = (jnp.sum(parts_ref[...], axis=0, keepdims=True)
                    + bias_ref[...])


_combine_call = pl.pallas_call(
    _combine_body,
    out_shape=jax.ShapeDtypeStruct((1, ACC), jnp.float32),
)


def kernel(inputs, kernel, bias, mask_rows, mask_cols):
    xt = inputs[:, :, 0].T
    w = kernel[:, 0]
    rows2d = mask_rows.reshape(NNZ // SEG, SEG)
    parts = _sc_call(xt, rows2d, mask_cols, w)
    bias_rep = jnp.broadcast_to(bias[:, 0:1], (OUT_LEN, BATCH))
    out_flat = _combine_call(parts, bias_rep.reshape(1, ACC))
    return out_flat.reshape(OUT_LEN, BATCH).T.reshape(BATCH, OUT_LEN, 1)
